# speedup vs baseline: 3.9322x; 3.9322x over previous
import functools

import jax
import jax.numpy as jnp
from jax import lax
from jax.experimental import pallas as pl
from jax.experimental.pallas import tpu as pltpu
from jax.experimental.pallas import tpu_sc as plsc

_NC = 2
_NS = 16
_LANES = 16


def kernel(tokens, word_table, learned_embedding):
    B, L = tokens.shape
    V, D = word_table.shape
    NVT = learned_embedding.shape[0]
    P = NVT + 3
    T = P + (L - 1)

    NW = _NC * _NS
    bpw = B // NW
    C1 = 128
    C2 = L - C1

    mesh = plsc.VectorSubcoreMesh(
        core_axis_name="c", subcore_axis_name="s",
        num_cores=_NC, num_subcores=_NS)

    @functools.partial(
        pl.kernel,
        out_type=jax.ShapeDtypeStruct((B, T, D), jnp.float32),
        mesh=mesh,
        scratch_types=[
            pltpu.VMEM((L,), jnp.int32),
            pltpu.VMEM((T, D), jnp.float32),
            pltpu.VMEM((8, D), jnp.float32),
            pltpu.SemaphoreType.DMA,
        ],
    )
    def sc_kernel(tokens_hbm, table_hbm, learned_hbm, out_hbm,
                  idx_v, buf, head_v, sem):
        wid = lax.axis_index("s") * _NC + lax.axis_index("c")
        b0 = wid * bpw

        pltpu.sync_copy(table_hbm.at[pl.ds(0, 8)], head_v)
        pltpu.sync_copy(learned_hbm, buf.at[pl.ds(2, NVT)])
        for l in range(D // _LANES):
            sl = pl.ds(l * _LANES, _LANES)
            buf[0, sl] = head_v[1, sl]
            buf[1, sl] = head_v[2, sl]

        def body(i, carry):
            b = b0 + i
            pltpu.sync_copy(tokens_hbm.at[b], idx_v)
            cp1 = pltpu.async_copy(
                table_hbm.at[idx_v.at[pl.ds(0, C1)]],
                buf.at[pl.ds(P - 1, C1)], sem)
            cp2 = pltpu.async_copy(
                table_hbm.at[idx_v.at[pl.ds(C1, C2)]],
                buf.at[pl.ds(P - 1 + C1, C2)], sem)
            cp1.wait()
            cp2.wait()
            for l in range(D // _LANES):
                sl = pl.ds(l * _LANES, _LANES)
                buf[P - 1, sl] = head_v[3, sl]
            pltpu.sync_copy(buf, out_hbm.at[b])
            return carry

        lax.fori_loop(0, bpw, body, 0)

    return sc_kernel(tokens, word_table, learned_embedding)


# speedup vs baseline: 4.7749x; 1.2143x over previous
import functools

import jax
import jax.numpy as jnp
from jax import lax
from jax.experimental import pallas as pl
from jax.experimental.pallas import tpu as pltpu
from jax.experimental.pallas import tpu_sc as plsc

_NC = 2
_NS = 16
_LANES = 16


def kernel(tokens, word_table, learned_embedding):
    B, L = tokens.shape
    V, D = word_table.shape
    NVT = learned_embedding.shape[0]
    P = NVT + 3
    T = P + (L - 1)

    NW = _NC * _NS
    bpw = B // NW
    nb2 = bpw // 2
    C1 = 128
    C2 = L - C1

    mesh = plsc.VectorSubcoreMesh(
        core_axis_name="c", subcore_axis_name="s",
        num_cores=_NC, num_subcores=_NS)

    @functools.partial(
        pl.kernel,
        out_type=jax.ShapeDtypeStruct((B, T, D), jnp.float32),
        mesh=mesh,
        scratch_types=[
            pltpu.VMEM((L,), jnp.int32),
            pltpu.VMEM((L,), jnp.int32),
            pltpu.VMEM((T, D), jnp.float32),
            pltpu.VMEM((T, D), jnp.float32),
            pltpu.VMEM((8, D), jnp.float32),
            pltpu.SemaphoreType.DMA,
            pltpu.SemaphoreType.DMA,
            pltpu.SemaphoreType.DMA,
            pltpu.SemaphoreType.DMA,
        ],
    )
    def sc_kernel(tokens_hbm, table_hbm, learned_hbm, out_hbm,
                  idx0, idx1, buf0, buf1, head_v,
                  gsem0, gsem1, wsem0, wsem1):
        wid = lax.axis_index("s") * _NC + lax.axis_index("c")
        b0 = wid * bpw

        def g_issue(buf, idx_v, gsem):
            pltpu.async_copy(table_hbm.at[idx_v.at[pl.ds(0, C1)]],
                             buf.at[pl.ds(P - 1, C1)], gsem)
            pltpu.async_copy(table_hbm.at[idx_v.at[pl.ds(C1, C2)]],
                             buf.at[pl.ds(P - 1 + C1, C2)], gsem)

        def g_wait(buf, idx_v, gsem):
            pltpu.make_async_copy(table_hbm.at[idx_v.at[pl.ds(0, C1)]],
                                  buf.at[pl.ds(P - 1, C1)], gsem).wait()
            pltpu.make_async_copy(table_hbm.at[idx_v.at[pl.ds(C1, C2)]],
                                  buf.at[pl.ds(P - 1 + C1, C2)], gsem).wait()

        def w_wait(buf, b, wsem):
            pltpu.make_async_copy(buf, out_hbm.at[b], wsem).wait()

        def patch_and_write(buf, b, wsem):
            for l in range(D // _LANES):
                sl = pl.ds(l * _LANES, _LANES)
                buf[P - 1, sl] = head_v[3, sl]
            pltpu.async_copy(buf, out_hbm.at[b], wsem)

        pltpu.sync_copy(table_hbm.at[pl.ds(0, 8)], head_v)
        pltpu.sync_copy(learned_hbm, buf0.at[pl.ds(2, NVT)])
        pltpu.sync_copy(learned_hbm, buf1.at[pl.ds(2, NVT)])
        for l in range(D // _LANES):
            sl = pl.ds(l * _LANES, _LANES)
            buf0[0, sl] = head_v[1, sl]
            buf0[1, sl] = head_v[2, sl]
            buf1[0, sl] = head_v[1, sl]
            buf1[1, sl] = head_v[2, sl]

        pltpu.sync_copy(tokens_hbm.at[b0], idx0)
        g_issue(buf0, idx0, gsem0)

        def body(j, carry):
            be = b0 + 2 * j
            bo = be + 1

            pltpu.sync_copy(tokens_hbm.at[bo], idx1)

            @pl.when(j > 0)
            def _():
                w_wait(buf1, bo - 2, wsem1)
            g_issue(buf1, idx1, gsem1)

            g_wait(buf0, idx0, gsem0)
            patch_and_write(buf0, be, wsem0)

            @pl.when(j < nb2 - 1)
            def _():
                pltpu.sync_copy(tokens_hbm.at[be + 2], idx0)
                w_wait(buf0, be, wsem0)
                g_issue(buf0, idx0, gsem0)

            g_wait(buf1, idx1, gsem1)
            patch_and_write(buf1, bo, wsem1)
            return carry

        lax.fori_loop(0, nb2, body, 0)

        w_wait(buf0, b0 + bpw - 2, wsem0)
        w_wait(buf1, b0 + bpw - 1, wsem1)

    return sc_kernel(tokens, word_table, learned_embedding)


# speedup vs baseline: 8.0654x; 1.6891x over previous
import functools

import jax
import jax.numpy as jnp
from jax import lax
from jax.experimental import pallas as pl
from jax.experimental.pallas import tpu as pltpu
from jax.experimental.pallas import tpu_sc as plsc

_NC = 2
_NS = 16
_LANES = 16


def kernel(tokens, word_table, learned_embedding):
    B, L = tokens.shape
    V, D = word_table.shape
    NVT = learned_embedding.shape[0]
    P = NVT + 3
    T = P + (L - 1)

    NW = _NC * _NS
    CH = 128
    NCH = B // CH
    TSTRIDE = NW // NCH
    NK = (T + TSTRIDE - 1) // TSTRIDE
    NKK = NK // 2
    NL = D // _LANES
    BC = 32

    mesh = plsc.VectorSubcoreMesh(
        core_axis_name="c", subcore_axis_name="s",
        num_cores=_NC, num_subcores=_NS)

    @functools.partial(
        pl.kernel,
        out_type=jax.ShapeDtypeStruct((T, B, D), jnp.float32),
        mesh=mesh,
        scratch_types=[
            pltpu.VMEM((NK, CH), jnp.int32),
            pltpu.VMEM((CH, D), jnp.float32),
            pltpu.VMEM((CH, D), jnp.float32),
            pltpu.VMEM((NVT + 4, D), jnp.float32),
            pltpu.VMEM((8, D), jnp.float32),
            pltpu.SemaphoreType.DMA,
            pltpu.SemaphoreType.DMA,
            pltpu.SemaphoreType.DMA,
            pltpu.SemaphoreType.DMA,
            pltpu.SemaphoreType.DMA,
        ],
    )
    def sc_kernel(tokt_hbm, table_hbm, learned_hbm, out_hbm,
                  idxall, buf0, buf1, combo_v, head_v,
                  isem, gsem0, gsem1, wsem0, wsem1):
        wid = lax.axis_index("s") * _NC + lax.axis_index("c")
        cb = (wid % NCH) * CH
        tq = wid // NCH

        def t_of(k):
            return tq + TSTRIDE * k

        def i_wait(k):
            pltpu.make_async_copy(tokt_hbm.at[0].at[pl.ds(cb, CH)],
                                  idxall.at[k], isem).wait()

        def issue(k, buf, gsem):
            t = t_of(k)
            i_wait(k)

            @pl.when(t < P)
            def _():
                r = jnp.where(t == 0, NVT,
                              jnp.where(t == 1, NVT + 1,
                                        jnp.where(t == P - 1, NVT + 2,
                                                  t - 2)))
                rows = [combo_v[r, pl.ds(l * _LANES, _LANES)]
                        for l in range(NL)]
                for i in range(BC):
                    for l in range(NL):
                        buf[i, pl.ds(l * _LANES, _LANES)] = rows[l]

            @pl.when(t >= P)
            def _():
                pltpu.async_copy(table_hbm.at[idxall.at[k]], buf, gsem)

        def g_wait(k, buf, gsem):
            @pl.when(t_of(k) >= P)
            def _():
                pltpu.make_async_copy(table_hbm.at[idxall.at[k]],
                                      buf, gsem).wait()

        def write(k, buf, wsem):
            t = t_of(k)

            @pl.when(t < P)
            def _():
                for q in range(CH // BC):
                    pltpu.async_copy(
                        buf.at[pl.ds(0, BC)],
                        out_hbm.at[t].at[pl.ds(cb + q * BC, BC)], wsem)

            @pl.when(t >= P)
            def _():
                pltpu.async_copy(buf, out_hbm.at[t].at[pl.ds(cb, CH)], wsem)

        def w_wait(k, buf, wsem):
            pltpu.make_async_copy(buf, out_hbm.at[t_of(k)].at[pl.ds(cb, CH)],
                                  wsem).wait()

        pltpu.sync_copy(learned_hbm, combo_v.at[pl.ds(0, NVT)])
        pltpu.sync_copy(table_hbm.at[pl.ds(0, 8)], head_v)
        for l in range(NL):
            sl = pl.ds(l * _LANES, _LANES)
            combo_v[NVT, sl] = head_v[1, sl]
            combo_v[NVT + 1, sl] = head_v[2, sl]
            combo_v[NVT + 2, sl] = head_v[3, sl]
        for k in range(NK):
            j = jnp.clip(t_of(k) - (P - 1), 0, L - 1)
            pltpu.async_copy(tokt_hbm.at[j].at[pl.ds(cb, CH)],
                             idxall.at[k], isem)

        issue(0, buf0, gsem0)

        def body(kk, carry):
            ke = 2 * kk
            ko = ke + 1
            ko_valid = t_of(ko) < T

            @pl.when(kk > 0)
            def _():
                w_wait(ko - 2, buf1, wsem1)

            @pl.when(ko_valid)
            def _():
                issue(ko, buf1, gsem1)

            g_wait(ke, buf0, gsem0)
            write(ke, buf0, wsem0)

            @pl.when(kk < NKK - 1)
            def _():
                w_wait(ke, buf0, wsem0)
                issue(ke + 2, buf0, gsem0)

            @pl.when(ko_valid)
            def _():
                g_wait(ko, buf1, gsem1)
                write(ko, buf1, wsem1)
            return carry

        lax.fori_loop(0, NKK, body, 0)

        w_wait(NK - 2, buf0, wsem0)

        @pl.when(t_of(NK - 1) < T)
        def _():
            w_wait(NK - 1, buf1, wsem1)

        @pl.when(t_of(NK - 1) >= T)
        def _():
            i_wait(NK - 1)

    tokens_t = tokens.T
    out = sc_kernel(tokens_t, word_table, learned_embedding)
    return jnp.transpose(out, (1, 0, 2))


# speedup vs baseline: 8.1638x; 1.0122x over previous
import functools

import jax
import jax.numpy as jnp
from jax import lax
from jax.experimental import pallas as pl
from jax.experimental.pallas import tpu as pltpu
from jax.experimental.pallas import tpu_sc as plsc

_NC = 2
_NS = 16
_LANES = 16


def kernel(tokens, word_table, learned_embedding):
    B, L = tokens.shape
    V, D = word_table.shape
    NVT = learned_embedding.shape[0]
    P = NVT + 3
    T = P + (L - 1)

    NW = _NC * _NS
    CH = 128
    NCH = B // CH
    TSTRIDE = NW // NCH
    NK = (T + TSTRIDE - 1) // TSTRIDE
    NBUF = 4
    NKK = NK // NBUF
    NL = D // _LANES
    BC = 32

    mesh = plsc.VectorSubcoreMesh(
        core_axis_name="c", subcore_axis_name="s",
        num_cores=_NC, num_subcores=_NS)

    @functools.partial(
        pl.kernel,
        out_type=jax.ShapeDtypeStruct((T, B, D), jnp.float32),
        mesh=mesh,
        scratch_types=[
            pltpu.VMEM((NK, CH), jnp.int32),
            [pltpu.VMEM((CH, D), jnp.float32) for _ in range(NBUF)],
            pltpu.VMEM((NVT + 4, D), jnp.float32),
            pltpu.VMEM((8, D), jnp.float32),
            pltpu.SemaphoreType.DMA,
            [pltpu.SemaphoreType.DMA for _ in range(NBUF)],
            [pltpu.SemaphoreType.DMA for _ in range(NBUF)],
        ],
    )
    def sc_kernel(tokt_hbm, table_hbm, learned_hbm, out_hbm,
                  idxall, bufs, combo_v, head_v,
                  isem, gsems, wsems):
        wid = lax.axis_index("s") * _NC + lax.axis_index("c")
        cb = (wid % NCH) * CH
        tq = wid // NCH

        def t_of(k):
            return tq + TSTRIDE * k

        def i_wait(k):
            pltpu.make_async_copy(tokt_hbm.at[0].at[pl.ds(cb, CH)],
                                  idxall.at[k], isem).wait()

        def issue(k, buf, gsem):
            t = t_of(k)
            i_wait(k)

            @pl.when(t < P)
            def _():
                r = jnp.where(t == 0, NVT,
                              jnp.where(t == 1, NVT + 1,
                                        jnp.where(t == P - 1, NVT + 2,
                                                  t - 2)))
                rows = [combo_v[r, pl.ds(l * _LANES, _LANES)]
                        for l in range(NL)]
                for i in range(BC):
                    for l in range(NL):
                        buf[i, pl.ds(l * _LANES, _LANES)] = rows[l]

            @pl.when(t >= P)
            def _():
                pltpu.async_copy(table_hbm.at[idxall.at[k]], buf, gsem)

        def g_wait(k, buf, gsem):
            @pl.when(t_of(k) >= P)
            def _():
                pltpu.make_async_copy(table_hbm.at[idxall.at[k]],
                                      buf, gsem).wait()

        def write(k, buf, wsem):
            t = t_of(k)

            @pl.when(t < P)
            def _():
                for q in range(CH // BC):
                    pltpu.async_copy(
                        buf.at[pl.ds(0, BC)],
                        out_hbm.at[t].at[pl.ds(cb + q * BC, BC)], wsem)

            @pl.when(t >= P)
            def _():
                pltpu.async_copy(buf, out_hbm.at[t].at[pl.ds(cb, CH)], wsem)

        def w_wait(k, buf, wsem):
            pltpu.make_async_copy(buf, out_hbm.at[t_of(k)].at[pl.ds(cb, CH)],
                                  wsem).wait()

        pltpu.sync_copy(learned_hbm, combo_v.at[pl.ds(0, NVT)])
        pltpu.sync_copy(table_hbm.at[pl.ds(0, 8)], head_v)
        for l in range(NL):
            sl = pl.ds(l * _LANES, _LANES)
            combo_v[NVT, sl] = head_v[1, sl]
            combo_v[NVT + 1, sl] = head_v[2, sl]
            combo_v[NVT + 2, sl] = head_v[3, sl]
        for k in range(NK):
            j = jnp.clip(t_of(k) - (P - 1), 0, L - 1)
            pltpu.async_copy(tokt_hbm.at[j].at[pl.ds(cb, CH)],
                             idxall.at[k], isem)

        issue(0, bufs[0], gsems[0])
        issue(1, bufs[1], gsems[1])

        def body(kk, carry):
            for p in range(NBUF):
                k = NBUF * kk + p
                pn = (p + 2) % NBUF

                @pl.when(k >= 2)
                def _():
                    w_wait(k - 2, bufs[pn], wsems[pn])

                @pl.when((k + 2 < NK) & (t_of(k + 2) < T))
                def _():
                    issue(k + 2, bufs[pn], gsems[pn])

                @pl.when(t_of(k) < T)
                def _():
                    g_wait(k, bufs[p], gsems[p])
                    write(k, bufs[p], wsems[p])
            return carry

        lax.fori_loop(0, NKK, body, 0)

        w_wait(NK - 2, bufs[(NK - 2) % NBUF], wsems[(NK - 2) % NBUF])

        @pl.when(t_of(NK - 1) < T)
        def _():
            w_wait(NK - 1, bufs[(NK - 1) % NBUF], wsems[(NK - 1) % NBUF])

        @pl.when(t_of(NK - 1) >= T)
        def _():
            i_wait(NK - 1)

    tokens_t = tokens.T
    out = sc_kernel(tokens_t, word_table, learned_embedding)
    return jnp.transpose(out, (1, 0, 2))


# speedup vs baseline: 8.7214x; 1.0683x over previous
import functools

import jax
import jax.numpy as jnp
from jax import lax
from jax.experimental import pallas as pl
from jax.experimental.pallas import tpu as pltpu
from jax.experimental.pallas import tpu_sc as plsc

_NC = 2
_NS = 16
_LANES = 16


def kernel(tokens, word_table, learned_embedding):
    B, L = tokens.shape
    V, D = word_table.shape
    NVT = learned_embedding.shape[0]
    P = NVT + 3
    T = P + (L - 1)
    G = L - 1

    NW = _NC * _NS
    CH = 128
    NCH = B // CH
    TSTRIDE = NW // NCH
    NK = (G + TSTRIDE - 1) // TSTRIDE
    NSLOT = 4 * ((NK + 3) // 4 + 1)

    mesh = plsc.VectorSubcoreMesh(
        core_axis_name="c", subcore_axis_name="s",
        num_cores=_NC, num_subcores=_NS)

    @functools.partial(
        pl.kernel,
        out_type=jax.ShapeDtypeStruct((T, B, D), jnp.float32),
        mesh=mesh,
        scratch_types=[
            pltpu.VMEM((NK, CH), jnp.int32),
            [pltpu.VMEM((CH, D), jnp.float32) for _ in range(4)],
            pltpu.SemaphoreType.DMA,
            [pltpu.SemaphoreType.DMA for _ in range(4)],
            [pltpu.SemaphoreType.DMA for _ in range(4)],
        ],
    )
    def sc_kernel(tokt_hbm, table_hbm, learned_hbm, out_hbm,
                  idxall, bufs, isem, gsems, wsems):
        wid = lax.axis_index("s") * _NC + lax.axis_index("c")
        cb = (wid % NCH) * CH
        tq = wid // NCH

        def t_of(k):
            return tq + TSTRIDE * k

        def i_wait(k):
            pltpu.make_async_copy(tokt_hbm.at[0].at[pl.ds(cb, CH)],
                                  idxall.at[k], isem).wait()

        def issue(k, buf, gsem):
            i_wait(k)
            pltpu.async_copy(table_hbm.at[idxall.at[k]], buf, gsem)

        def g_wait(k, buf, gsem):
            pltpu.make_async_copy(table_hbm.at[idxall.at[k]],
                                  buf, gsem).wait()

        def write(k, buf, wsem):
            pltpu.async_copy(buf, out_hbm.at[P + t_of(k)].at[pl.ds(cb, CH)],
                             wsem)

        def w_wait(k, buf, wsem):
            pltpu.make_async_copy(buf,
                                  out_hbm.at[P + t_of(k)].at[pl.ds(cb, CH)],
                                  wsem).wait()

        for k in range(NK):
            j = jnp.clip(t_of(k) + 1, 0, L - 1)
            pltpu.async_copy(tokt_hbm.at[j].at[pl.ds(cb, CH)],
                             idxall.at[k], isem)

        issue(0, bufs[0], gsems[0])
        issue(1, bufs[1], gsems[1])

        def body(kk, carry):
            for p in range(4):
                k = 4 * kk + p
                pn = (p + 2) % 4

                @pl.when((k >= 2) & (t_of(jnp.maximum(k - 2, 0)) < G))
                def _():
                    w_wait(k - 2, bufs[pn], wsems[pn])

                @pl.when((k + 2 < NK) & (t_of(k + 2) < G))
                def _():
                    issue(k + 2, bufs[pn], gsems[pn])

                @pl.when(t_of(k) < G)
                def _():
                    g_wait(k, bufs[p], gsems[p])
                    write(k, bufs[p], wsems[p])
            return carry

        lax.fori_loop(0, NSLOT // 4, body, 0)

        @pl.when(t_of(NK - 1) >= G)
        def _():
            i_wait(NK - 1)

    tokens_t = tokens.T
    out = sc_kernel(tokens_t, word_table, learned_embedding)

    learned_block = jnp.concatenate(
        [word_table[1:2], word_table[2:3], learned_embedding,
         word_table[3:4]], axis=0)
    lb = jnp.broadcast_to(learned_block[:, None, :], (P, B, D))
    full = lax.dynamic_update_slice(out, lb, (0, 0, 0))
    return jnp.transpose(full, (1, 0, 2))


# speedup vs baseline: 8.7667x; 1.0052x over previous
import functools

import jax
import jax.numpy as jnp
from jax import lax
from jax.experimental import pallas as pl
from jax.experimental.pallas import tpu as pltpu
from jax.experimental.pallas import tpu_sc as plsc

_NC = 2
_NS = 16
_LANES = 16


def kernel(tokens, word_table, learned_embedding):
    B, L = tokens.shape
    V, D = word_table.shape
    NVT = learned_embedding.shape[0]
    P = NVT + 3
    T = P + (L - 1)
    G = L - 1

    NW = _NC * _NS
    CH = 128
    NCH = B // CH
    TSTRIDE = NW // NCH
    NK = (G + TSTRIDE - 1) // TSTRIDE
    NBUF = 6
    LAG = NBUF // 2
    NSLOT = NBUF * ((NK + LAG + NBUF - 1) // NBUF)

    mesh = plsc.VectorSubcoreMesh(
        core_axis_name="c", subcore_axis_name="s",
        num_cores=_NC, num_subcores=_NS)

    @functools.partial(
        pl.kernel,
        out_type=jax.ShapeDtypeStruct((T, B, D), jnp.float32),
        mesh=mesh,
        scratch_types=[
            pltpu.VMEM((NK, CH), jnp.int32),
            [pltpu.VMEM((CH, D), jnp.float32) for _ in range(NBUF)],
            pltpu.SemaphoreType.DMA,
            [pltpu.SemaphoreType.DMA for _ in range(NBUF)],
            [pltpu.SemaphoreType.DMA for _ in range(NBUF)],
        ],
    )
    def sc_kernel(tokt_hbm, table_hbm, learned_hbm, out_hbm,
                  idxall, bufs, isem, gsems, wsems):
        wid = lax.axis_index("s") * _NC + lax.axis_index("c")
        cb = (wid % NCH) * CH
        tq = wid // NCH

        def t_of(k):
            return tq + TSTRIDE * k

        def i_wait(k):
            pltpu.make_async_copy(tokt_hbm.at[0].at[pl.ds(cb, CH)],
                                  idxall.at[k], isem).wait()

        def issue(k, buf, gsem):
            i_wait(k)
            pltpu.async_copy(table_hbm.at[idxall.at[k]], buf, gsem)

        def g_wait(k, buf, gsem):
            pltpu.make_async_copy(table_hbm.at[idxall.at[k]],
                                  buf, gsem).wait()

        def write(k, buf, wsem):
            pltpu.async_copy(buf, out_hbm.at[P + t_of(k)].at[pl.ds(cb, CH)],
                             wsem)

        def w_wait(k, buf, wsem):
            pltpu.make_async_copy(buf,
                                  out_hbm.at[P + t_of(k)].at[pl.ds(cb, CH)],
                                  wsem).wait()

        for k in range(LAG):
            j = jnp.clip(t_of(k) + 1, 0, L - 1)
            pltpu.async_copy(tokt_hbm.at[j].at[pl.ds(cb, CH)],
                             idxall.at[k], isem)
        for k in range(LAG):
            issue(k, bufs[k], gsems[k])
        for k in range(LAG, NK):
            j = jnp.clip(t_of(k) + 1, 0, L - 1)
            pltpu.async_copy(tokt_hbm.at[j].at[pl.ds(cb, CH)],
                             idxall.at[k], isem)

        def body(kk, carry):
            for p in range(NBUF):
                k = NBUF * kk + p
                pn = (p + LAG) % NBUF

                @pl.when((k >= LAG) & (t_of(jnp.maximum(k - LAG, 0)) < G))
                def _():
                    w_wait(k - LAG, bufs[pn], wsems[pn])

                @pl.when((k + LAG < NK) & (t_of(k + LAG) < G))
                def _():
                    issue(k + LAG, bufs[pn], gsems[pn])

                @pl.when(t_of(k) < G)
                def _():
                    g_wait(k, bufs[p], gsems[p])
                    write(k, bufs[p], wsems[p])
            return carry

        lax.fori_loop(0, NSLOT // NBUF, body, 0)

        @pl.when(t_of(NK - 1) >= G)
        def _():
            i_wait(NK - 1)

    tokens_t = tokens.T
    out = sc_kernel(tokens_t, word_table, learned_embedding)

    learned_block = jnp.concatenate(
        [word_table[1:2], word_table[2:3], learned_embedding,
         word_table[3:4]], axis=0)
    lb = jnp.broadcast_to(learned_block[:, None, :], (P, B, D))
    full = lax.dynamic_update_slice(out, lb, (0, 0, 0))
    return jnp.transpose(full, (1, 0, 2))
